# baseline (speedup 1.0000x reference)
import jax
import jax.numpy as jnp
from jax.experimental import pallas as pl

N_TOK = 16384
EMBED = 64
N_EXP = 16
LABEL = 64
HIDDEN = EMBED // 2

BLOCK = 2048
OUTW = 256


def _moe_block(x_ref, noise_ref, w_gate_ref, w_noise_ref, w1_ref, b1_ref,
               w2_ref, out_ref):
    x = x_ref[...]
    f32 = jnp.float32
    B = x.shape[0]

    clean = jnp.dot(x, w_gate_ref[...], preferred_element_type=f32)
    raw = jnp.dot(x, w_noise_ref[...], preferred_element_type=f32)
    noise_std = jax.nn.softplus(raw) + 1e-2
    logits = clean + noise_ref[...] * noise_std

    col = jax.lax.broadcasted_iota(jnp.int32, logits.shape, 1)
    big = jnp.int32(N_EXP)
    v1 = jnp.max(logits, axis=1, keepdims=True)
    idx1 = jnp.min(jnp.where(logits == v1, col, big), axis=1, keepdims=True)
    masked = jnp.where(col == idx1, -jnp.inf, logits)
    v2 = jnp.max(masked, axis=1, keepdims=True)
    idx2 = jnp.min(jnp.where(masked == v2, col, big), axis=1, keepdims=True)
    e2 = jnp.exp(v2 - v1)
    g1 = 1.0 / (1.0 + e2)
    g2 = e2 / (1.0 + e2)

    xb = x.astype(jnp.bfloat16)
    h = jnp.tanh(jnp.dot(xb, w1_ref[...], preferred_element_type=f32)
                 + b1_ref[...])
    hb = h.astype(jnp.bfloat16)

    ecol = jax.lax.broadcasted_iota(jnp.int32, h.shape, 1) // HIDDEN
    zb = jnp.bfloat16(0)
    s1 = jnp.where(ecol == idx1, hb, zb)
    s2 = jnp.where(ecol == idx2, hb, zb)
    oh1 = jnp.where(col == idx1, 1.0, 0.0).astype(jnp.bfloat16)
    oh2 = jnp.where(col == idx2, 1.0, 0.0).astype(jnp.bfloat16)

    lhs = jnp.concatenate([s1, s2, oh1, oh2], axis=1)
    pfull = jnp.dot(lhs, w2_ref[...], preferred_element_type=f32)

    lane = jax.lax.broadcasted_iota(jnp.int32, (B, 2 * LABEL), 1)
    gcol = jnp.where(lane < LABEL, g1, g2)
    meta = jnp.where(lane == 0, idx1.astype(f32),
                     jnp.where(lane == 1, idx2.astype(f32), 0.0))
    out_ref[...] = jnp.concatenate([pfull * gcol, meta], axis=1)


@jax.jit
def kernel(x, noise, w_gate, w_noise, fc1_w, fc1_b, fc2_w, fc2_b):
    w1 = fc1_w.reshape(N_EXP * HIDDEN, EMBED).T.astype(jnp.bfloat16)
    b1 = fc1_b.reshape(1, N_EXP * HIDDEN)

    w2s = jnp.transpose(fc2_w, (0, 2, 1)).reshape(N_EXP * HIDDEN, LABEL)
    zw = jnp.zeros_like(w2s)
    z = jnp.zeros_like(fc2_b)
    w2 = jnp.concatenate([
        jnp.concatenate([w2s, zw], axis=1),
        jnp.concatenate([zw, w2s], axis=1),
        jnp.concatenate([fc2_b, z], axis=1),
        jnp.concatenate([z, fc2_b], axis=1),
    ], axis=0).astype(jnp.bfloat16)

    grid = (N_TOK // BLOCK,)
    dat = pl.pallas_call(
        _moe_block,
        grid=grid,
        in_specs=[
            pl.BlockSpec((BLOCK, EMBED), lambda i: (i, 0)),
            pl.BlockSpec((BLOCK, N_EXP), lambda i: (i, 0)),
            pl.BlockSpec((EMBED, N_EXP), lambda i: (0, 0)),
            pl.BlockSpec((EMBED, N_EXP), lambda i: (0, 0)),
            pl.BlockSpec((EMBED, N_EXP * HIDDEN), lambda i: (0, 0)),
            pl.BlockSpec((1, N_EXP * HIDDEN), lambda i: (0, 0)),
            pl.BlockSpec((2 * N_EXP * (HIDDEN + 1), 2 * LABEL),
                         lambda i: (0, 0)),
        ],
        out_specs=pl.BlockSpec((BLOCK, OUTW), lambda i: (i, 0)),
        out_shape=jax.ShapeDtypeStruct((N_TOK, OUTW), jnp.float32),
    )(x, noise, w_gate, w_noise, w1, b1, w2)

    d1 = dat[:, 0:LABEL][:, None, :]
    d2 = dat[:, LABEL:2 * LABEL][:, None, :]
    i1 = dat[:, 2 * LABEL:2 * LABEL + 1][:, :, None]
    i2 = dat[:, 2 * LABEL + 1:2 * LABEL + 2][:, :, None]
    eid = jnp.arange(N_EXP, dtype=jnp.float32)[None, :, None]
    out = jnp.where(eid == i1, d1, 0.0) + jnp.where(eid == i2, d2, 0.0)
    return out
